# baseline (speedup 1.0000x reference)
import functools

import jax
import jax.numpy as jnp
from jax import lax
from jax.experimental import pallas as pl
from jax.experimental.pallas import tpu as pltpu
from jax.experimental.pallas import tpu_sc as plsc

_N = 50000
_E = 1600000
_B = 128

_NP = 53248
_NPR = _NP // 128
_SLAB = _NP // 16
_EP = 1605632
_EPT = _EP // 32
_CHE = 512
_NCH = _EPT // _CHE
_SENT = _N
_NBLK = _NP // 4096


def _mesh():
    return plsc.VectorSubcoreMesh(core_axis_name="c", subcore_axis_name="s")


def _stage_in(hbm, shared, stage, slab):
    pltpu.sync_copy(hbm.at[pl.ds(slab, _SLAB)], stage)
    pltpu.sync_copy(stage, shared.at[pl.ds(slab, _SLAB)])


def _stage_out(shared, hbm_slice, stage, slab):
    pltpu.sync_copy(shared.at[pl.ds(slab, _SLAB)], stage)
    pltpu.sync_copy(stage, hbm_slice)


def _sc_degree(dst0, dst1, zeros_np):
    @functools.partial(
        pl.kernel,
        out_type=jax.ShapeDtypeStruct((2, 2, _NP), jnp.float32),
        mesh=_mesh(),
        scratch_types=[
            pltpu.VMEM((_CHE,), jnp.int32),
            pltpu.VMEM((_CHE,), jnp.float32),
            pltpu.VMEM((_SLAB,), jnp.float32),
            pltpu.VMEM_SHARED((_NP,), jnp.float32),
            pltpu.VMEM_SHARED((_NP,), jnp.float32),
        ],
    )
    def k(d0_h, d1_h, z_h, out_h, idx_v, ones_v, stage_v, acc0_sh, acc1_sh):
        c = lax.axis_index("c")
        s = lax.axis_index("s")
        slab = s * _SLAB
        pltpu.sync_copy(z_h.at[pl.ds(slab, _SLAB)], stage_v)
        pltpu.sync_copy(stage_v, acc0_sh.at[pl.ds(slab, _SLAB)])
        pltpu.sync_copy(stage_v, acc1_sh.at[pl.ds(slab, _SLAB)])
        for l in range(_CHE // 16):
            ones_v[pl.ds(l * 16, 16)] = jnp.full((16,), 1.0, jnp.float32)
        plsc.subcore_barrier()
        e0 = (s * 2 + c) * _EPT
        for d_h, acc in ((d0_h, acc0_sh), (d1_h, acc1_sh)):
            def body(kk, carry, d_h=d_h, acc=acc):
                pltpu.sync_copy(d_h.at[pl.ds(e0 + kk * _CHE, _CHE)], idx_v)
                pltpu.sync_copy(ones_v, acc.at[idx_v], add=True)
                return carry
            lax.fori_loop(0, _NCH, body, 0)
        plsc.subcore_barrier()
        _stage_out(acc0_sh, out_h.at[0, c, pl.ds(slab, _SLAB)], stage_v, slab)
        _stage_out(acc1_sh, out_h.at[1, c, pl.ds(slab, _SLAB)], stage_v, slab)

    return k(dst0, dst1, zeros_np)


def _sc_gsum(src0, dst0, src1, dst1, y0, y1, zeros_np):
    @functools.partial(
        pl.kernel,
        out_type=jax.ShapeDtypeStruct((2, 2, _NP), jnp.float32),
        mesh=_mesh(),
        scratch_types=[
            pltpu.VMEM((_CHE,), jnp.int32),
            pltpu.VMEM((_CHE,), jnp.int32),
            pltpu.VMEM((_CHE,), jnp.float32),
            pltpu.VMEM((_SLAB,), jnp.float32),
            pltpu.SemaphoreType.DMA,
            pltpu.SemaphoreType.DMA,
            pltpu.VMEM_SHARED((_NP,), jnp.float32),
            pltpu.VMEM_SHARED((_NP,), jnp.float32),
            pltpu.VMEM_SHARED((_NP,), jnp.float32),
            pltpu.VMEM_SHARED((_NP,), jnp.float32),
        ],
    )
    def k(s0_h, d0_h, s1_h, d1_h, y0_h, y1_h, z_h, out_h,
          idxs_v, idxd_v, ybuf_v, stage_v, sem0, sem1,
          y0_sh, y1_sh, acc0_sh, acc1_sh):
        c = lax.axis_index("c")
        s = lax.axis_index("s")
        slab = s * _SLAB
        pltpu.sync_copy(z_h.at[pl.ds(slab, _SLAB)], stage_v)
        pltpu.sync_copy(stage_v, acc0_sh.at[pl.ds(slab, _SLAB)])
        pltpu.sync_copy(stage_v, acc1_sh.at[pl.ds(slab, _SLAB)])
        _stage_in(y0_h, y0_sh, stage_v, slab)
        _stage_in(y1_h, y1_sh, stage_v, slab)
        plsc.subcore_barrier()
        e0 = (s * 2 + c) * _EPT
        for s_h, d_h, y_sh, acc in ((s0_h, d0_h, y0_sh, acc0_sh),
                                    (s1_h, d1_h, y1_sh, acc1_sh)):
            def body(kk, carry, s_h=s_h, d_h=d_h, y_sh=y_sh, acc=acc):
                base = e0 + kk * _CHE
                ci = pltpu.async_copy(s_h.at[pl.ds(base, _CHE)], idxs_v, sem0)
                cd = pltpu.async_copy(d_h.at[pl.ds(base, _CHE)], idxd_v, sem1)
                ci.wait()
                cd.wait()
                pltpu.sync_copy(y_sh.at[idxs_v], ybuf_v)
                pltpu.sync_copy(ybuf_v, acc.at[idxd_v], add=True)
                return carry
            lax.fori_loop(0, _NCH, body, 0)
        plsc.subcore_barrier()
        _stage_out(acc0_sh, out_h.at[0, c, pl.ds(slab, _SLAB)], stage_v, slab)
        _stage_out(acc1_sh, out_h.at[1, c, pl.ds(slab, _SLAB)], stage_v, slab)

    return k(src0, dst0, src1, dst1, y0, y1, zeros_np)


def _sc_att(src0, dst0, src1, dst1, as0, as1, ad0, ad1, g0, g1, zeros_np):
    @functools.partial(
        pl.kernel,
        out_type=[jax.ShapeDtypeStruct((2, 2, _NP), jnp.float32),
                  jax.ShapeDtypeStruct((2, 2, _NP), jnp.float32)],
        mesh=_mesh(),
        scratch_types=[
            pltpu.VMEM((_CHE,), jnp.int32),
            pltpu.VMEM((_CHE,), jnp.int32),
            pltpu.VMEM((_CHE,), jnp.float32),
            pltpu.VMEM((_CHE,), jnp.float32),
            pltpu.VMEM((_CHE,), jnp.float32),
            pltpu.VMEM((_CHE,), jnp.float32),
            pltpu.VMEM((_CHE,), jnp.float32),
            pltpu.VMEM((_SLAB,), jnp.float32),
            pltpu.SemaphoreType.DMA,
            pltpu.SemaphoreType.DMA,
            pltpu.SemaphoreType.DMA,
            pltpu.VMEM_SHARED((_NP,), jnp.float32),
            pltpu.VMEM_SHARED((_NP,), jnp.float32),
            pltpu.VMEM_SHARED((_NP,), jnp.float32),
            pltpu.VMEM_SHARED((_NP,), jnp.float32),
            pltpu.VMEM_SHARED((_NP,), jnp.float32),
            pltpu.VMEM_SHARED((_NP,), jnp.float32),
            pltpu.VMEM_SHARED((_NP,), jnp.float32),
            pltpu.VMEM_SHARED((_NP,), jnp.float32),
            pltpu.VMEM_SHARED((_NP,), jnp.float32),
            pltpu.VMEM_SHARED((_NP,), jnp.float32),
        ],
    )
    def k(s0_h, d0_h, s1_h, d1_h, as0_h, as1_h, ad0_h, ad1_h, g0_h, g1_h, z_h,
          zout_h, nout_h,
          idxs_v, idxd_v, asb_v, adb_v, gb_v, tb_v, ub_v, stage_v,
          sem0, sem1, sem2,
          as0_sh, as1_sh, ad0_sh, ad1_sh, g0_sh, g1_sh,
          zac0_sh, zac1_sh, nac0_sh, nac1_sh):
        c = lax.axis_index("c")
        s = lax.axis_index("s")
        slab = s * _SLAB
        pltpu.sync_copy(z_h.at[pl.ds(slab, _SLAB)], stage_v)
        for acc in (zac0_sh, zac1_sh, nac0_sh, nac1_sh):
            pltpu.sync_copy(stage_v, acc.at[pl.ds(slab, _SLAB)])
        for hbm, sh in ((as0_h, as0_sh), (as1_h, as1_sh), (ad0_h, ad0_sh),
                        (ad1_h, ad1_sh), (g0_h, g0_sh), (g1_h, g1_sh)):
            _stage_in(hbm, sh, stage_v, slab)
        plsc.subcore_barrier()
        e0 = (s * 2 + c) * _EPT
        for s_h, d_h, as_sh, ad_sh, g_sh, zac, nac in (
                (s0_h, d0_h, as0_sh, ad0_sh, g0_sh, zac0_sh, nac0_sh),
                (s1_h, d1_h, as1_sh, ad1_sh, g1_sh, zac1_sh, nac1_sh)):
            def body(kk, carry, s_h=s_h, d_h=d_h, as_sh=as_sh, ad_sh=ad_sh,
                     g_sh=g_sh, zac=zac, nac=nac):
                base = e0 + kk * _CHE
                ci = pltpu.async_copy(s_h.at[pl.ds(base, _CHE)], idxs_v, sem0)
                cd = pltpu.async_copy(d_h.at[pl.ds(base, _CHE)], idxd_v, sem1)
                ci.wait()
                cd.wait()
                g1c = pltpu.async_copy(as_sh.at[idxs_v], asb_v, sem0)
                g2c = pltpu.async_copy(ad_sh.at[idxd_v], adb_v, sem1)
                g3c = pltpu.async_copy(g_sh.at[idxs_v], gb_v, sem2)
                g1c.wait()
                g2c.wait()
                g3c.wait()
                for l in range(_CHE // 16):
                    sl = pl.ds(l * 16, 16)
                    e = asb_v[sl] + adb_v[sl]
                    e = jnp.where(e > 0, e, e * 0.2)
                    t = jnp.exp(e)
                    tb_v[sl] = t
                    ub_v[sl] = t * gb_v[sl]
                s1c = pltpu.async_copy(tb_v, zac.at[idxd_v], sem0, add=True)
                s2c = pltpu.async_copy(ub_v, nac.at[idxd_v], sem1, add=True)
                s1c.wait()
                s2c.wait()
                return carry
            lax.fori_loop(0, _NCH, body, 0)
        plsc.subcore_barrier()
        _stage_out(zac0_sh, zout_h.at[0, c, pl.ds(slab, _SLAB)], stage_v, slab)
        _stage_out(zac1_sh, zout_h.at[1, c, pl.ds(slab, _SLAB)], stage_v, slab)
        _stage_out(nac0_sh, nout_h.at[0, c, pl.ds(slab, _SLAB)], stage_v, slab)
        _stage_out(nac1_sh, nout_h.at[1, c, pl.ds(slab, _SLAB)], stage_v, slab)

    return k(src0, dst0, src1, dst1, as0, as1, ad0, ad1, g0, g1, zeros_np)


def _tc_prep(x2, degp):
    def body(x_ref, d_ref, y_ref, di_ref):
        d = d_ref[:, 0] + d_ref[:, 1] + 1.0
        di = lax.rsqrt(d)
        di_ref[...] = di
        y_ref[...] = x_ref[...] * di

    return pl.pallas_call(
        body,
        out_shape=[jax.ShapeDtypeStruct((2, _NPR, 128), jnp.float32),
                   jax.ShapeDtypeStruct((2, _NPR, 128), jnp.float32)],
    )(x2, degp)


def _tc_node(gsump, y2, di2, scal):
    def body(gp_ref, y_ref, di_ref, sc_ref, as_ref, ad_ref, g_ref, ts_ref):
        ps, qs, pd, qd = sc_ref[0], sc_ref[1], sc_ref[2], sc_ref[3]
        g = di_ref[...] * (y_ref[...] + gp_ref[:, 0] + gp_ref[:, 1])
        a_s = g * ps + qs
        a_d = g * pd + qd
        e = a_s + a_d
        e = jnp.where(e > 0, e, e * 0.2)
        as_ref[...] = a_s
        ad_ref[...] = a_d
        g_ref[...] = g
        ts_ref[...] = jnp.exp(e)

    return pl.pallas_call(
        body,
        in_specs=[pl.BlockSpec(memory_space=pltpu.MemorySpace.VMEM),
                  pl.BlockSpec(memory_space=pltpu.MemorySpace.VMEM),
                  pl.BlockSpec(memory_space=pltpu.MemorySpace.VMEM),
                  pl.BlockSpec(memory_space=pltpu.MemorySpace.SMEM)],
        out_shape=[jax.ShapeDtypeStruct((2, _NPR, 128), jnp.float32)] * 4,
    )(gsump, y2, di2, scal)


def _tc_final(ts2, g2, z0, z1, n0, n1, ids2, wcols):
    p1, q1, r1, pg, qg, rg, W2, b2, W3, b3, cW1, cb1, cW2, cb2, cW3, cb3 = wcols
    nsteps = 2 * _NBLK

    def body(ts_ref, g_ref, z0_ref, z1_ref, n0_ref, n1_ref, id_ref,
             p1_ref, q1_ref, r1_ref, pg_ref, qg_ref, rg_ref,
             W2_ref, b2_ref, W3_ref, b3_ref,
             cW1_ref, cb1_ref, cW2_ref, cb2_ref, cW3_ref, cb3_ref,
             out_ref, accs, acct):
        i = pl.program_id(0)

        @pl.when(i == 0)
        def _init():
            accs[...] = jnp.zeros((56, 128), jnp.float32)
            acct[...] = jnp.zeros((56, 128), jnp.float32)

        ts = ts_ref[0].reshape(1, 4096)
        g = g_ref[0].reshape(1, 4096)
        z = ts + z0_ref[0].reshape(1, 4096) + z1_ref[0].reshape(1, 4096)
        nm = ts * g + n0_ref[0].reshape(1, 4096) + n1_ref[0].reshape(1, 4096)
        A = nm / (z + 1e-16)
        S = z / (z + 1e-16)
        st = A * p1_ref[...] + S * q1_ref[...] + r1_ref[...]
        st = jnp.where(st > 0, st, st * 0.01)
        gt = A * pg_ref[...] + S * qg_ref[...] + rg_ref[...]
        gt = gt - jnp.max(gt, axis=0, keepdims=True)
        gt = jnp.exp(gt)
        gt = gt / jnp.sum(gt, axis=0, keepdims=True)
        sg = st * gt
        s48 = lax.dot_general(W2_ref[...], sg, (((0,), (0,)), ((), ())),
                              preferred_element_type=jnp.float32)
        s48 = s48 + b2_ref[...]
        s48 = jnp.where(s48 > 0, s48, s48 * 0.01)
        s49 = jnp.concatenate([s48, jnp.ones((1, 4096), jnp.float32)], axis=0)
        ids = id_ref[0].reshape(1, 4096)
        bvec = lax.broadcasted_iota(jnp.int32, (_B, 1), 0)
        oh = (ids == bvec).astype(jnp.float32)
        contrib = lax.dot_general(s49, oh, (((1,), (1,)), ((), ())),
                                  preferred_element_type=jnp.float32)

        @pl.when(i < _NBLK)
        def _adds():
            accs[0:48, :] = accs[0:48, :] + contrib[0:48, :]
            accs[48:56, :] = accs[48:56, :] + contrib[48:49, :]

        @pl.when(i >= _NBLK)
        def _addt():
            acct[0:48, :] = acct[0:48, :] + contrib[0:48, :]
            acct[48:56, :] = acct[48:56, :] + contrib[48:49, :]

        @pl.when(i == nsteps - 1)
        def _cls():
            def emb(acc):
                cnt = acc[48:56, :][0:1, :]
                mean = acc[0:48, :] / jnp.maximum(cnt, 1.0)
                e = lax.dot_general(W3_ref[...], mean, (((0,), (0,)), ((), ())),
                                    preferred_element_type=jnp.float32)
                e = e + b3_ref[...]
                return jnp.where(cnt > 0, e, 0.0)

            cat = jnp.concatenate([emb(accs[...]), emb(acct[...])], axis=0)
            h = lax.dot_general(cW1_ref[...], cat, (((0,), (0,)), ((), ())),
                                preferred_element_type=jnp.float32) + cb1_ref[...]
            h = jnp.maximum(h, 0.0)
            h = lax.dot_general(cW2_ref[...], h, (((0,), (0,)), ((), ())),
                                preferred_element_type=jnp.float32) + cb2_ref[...]
            h = jnp.maximum(h, 0.0)
            o = lax.dot_general(cW3_ref[...], h, (((0,), (0,)), ((), ())),
                                preferred_element_type=jnp.float32) + cb3_ref[...]
            out_ref[...] = o

    node_spec = pl.BlockSpec((1, 32, 128), lambda i: (i, 0, 0))
    full = lambda shape: pl.BlockSpec(shape, lambda i: tuple(0 for _ in shape))
    return pl.pallas_call(
        body,
        grid=(nsteps,),
        in_specs=[node_spec, node_spec, node_spec, node_spec, node_spec,
                  node_spec, node_spec,
                  full((32, 1)), full((32, 1)), full((32, 1)),
                  full((32, 1)), full((32, 1)), full((32, 1)),
                  full((32, 48)), full((48, 1)), full((48, 32)), full((32, 1)),
                  full((64, 32)), full((32, 1)), full((32, 32)), full((32, 1)),
                  full((32, 2)), full((2, 1))],
        out_specs=pl.BlockSpec((2, 128), lambda i: (0, 0)),
        out_shape=jax.ShapeDtypeStruct((2, 128), jnp.float32),
        scratch_shapes=[pltpu.VMEM((56, 128), jnp.float32),
                        pltpu.VMEM((56, 128), jnp.float32)],
    )(ts2, g2, z0, z1, n0, n1, ids2,
      p1, q1, r1, pg, qg, rg, W2, b2, W3, b3, cW1, cb1, cW2, cb2, cW3, cb3)


def _pad_nodes(v):
    return jnp.concatenate([v, jnp.zeros((_NP - _N,), v.dtype)])


def _pad_edges(v):
    return jnp.concatenate(
        [v.astype(jnp.int32),
         jnp.full((_EP - _E,), _SENT, jnp.int32)])


def kernel(x_s, x_t, edge_index_s, edge_index_t, x_s_batch, x_t_batch,
           gcn_W, gcn_b, gat_W, gat_asrc, gat_adst, gat_b,
           agg_W1, agg_b1, agg_Wg, agg_bg, agg_W2, agg_b2, agg_W3, agg_b3,
           cls_W1, cls_b1, cls_W2, cls_b2, cls_W3, cls_b3):
    f32 = jnp.float32
    w = gcn_W[0]
    u = w @ gat_W
    cv = gcn_b @ gat_W
    scal = jnp.stack([u @ gat_asrc, cv @ gat_asrc,
                      u @ gat_adst, cv @ gat_adst]).astype(f32)
    col = lambda v: v.astype(f32)[:, None]
    wcols = (col(u @ agg_W1), col(cv @ agg_W1), col(gat_b @ agg_W1 + agg_b1),
             col(u @ agg_Wg), col(cv @ agg_Wg), col(gat_b @ agg_Wg + agg_bg),
             agg_W2.astype(f32), col(agg_b2), agg_W3.astype(f32), col(agg_b3),
             cls_W1.astype(f32), col(cls_b1), cls_W2.astype(f32), col(cls_b2),
             cls_W3.astype(f32), col(cls_b3))

    x2 = jnp.stack([_pad_nodes(x_s[:, 0]), _pad_nodes(x_t[:, 0])])
    x2 = x2.reshape(2, _NPR, 128)
    src0 = _pad_edges(edge_index_s[0])
    dst0 = _pad_edges(edge_index_s[1])
    src1 = _pad_edges(edge_index_t[0])
    dst1 = _pad_edges(edge_index_t[1])
    pad_ids = jnp.full((_NP - _N,), _B, jnp.int32)
    ids2 = jnp.stack([
        jnp.concatenate([x_s_batch.astype(jnp.int32), pad_ids]),
        jnp.concatenate([x_t_batch.astype(jnp.int32), pad_ids]),
    ]).reshape(2 * _NBLK, 32, 128)
    zeros_np = jnp.zeros((_NP,), f32)

    degp = _sc_degree(dst0, dst1, zeros_np)
    y2, di2 = _tc_prep(x2, degp.reshape(2, 2, _NPR, 128))
    y0 = y2[0].reshape(_NP)
    y1 = y2[1].reshape(_NP)
    gsump = _sc_gsum(src0, dst0, src1, dst1, y0, y1, zeros_np)
    as2, ad2, g2, ts2 = _tc_node(gsump.reshape(2, 2, _NPR, 128), y2, di2, scal)
    zp, nump = _sc_att(src0, dst0, src1, dst1,
                       as2[0].reshape(_NP), as2[1].reshape(_NP),
                       ad2[0].reshape(_NP), ad2[1].reshape(_NP),
                       g2[0].reshape(_NP), g2[1].reshape(_NP), zeros_np)
    blk = lambda a: a.reshape(2 * _NBLK, 32, 128)
    out = _tc_final(blk(ts2), blk(g2),
                    blk(zp[:, 0]), blk(zp[:, 1]),
                    blk(nump[:, 0]), blk(nump[:, 1]), ids2, wcols)
    return out.T


# speedup vs baseline: 1.4929x; 1.4929x over previous
import functools

import jax
import jax.numpy as jnp
from jax import lax
from jax.experimental import pallas as pl
from jax.experimental.pallas import tpu as pltpu
from jax.experimental.pallas import tpu_sc as plsc

_N = 50000
_E = 1600000
_B = 128

_NP = 53248
_NPR = _NP // 128
_SLAB = _NP // 16
_EP = 1605632
_EPT = _EP // 32
_CHE = 1024
_NCH = _EPT // _CHE
_SENT = _N
_NBLK = _NP // 4096


def _mesh():
    return plsc.VectorSubcoreMesh(core_axis_name="c", subcore_axis_name="s")


def _stage_in(hbm, shared, stage, slab):
    pltpu.sync_copy(hbm.at[pl.ds(slab, _SLAB)], stage)
    pltpu.sync_copy(stage, shared.at[pl.ds(slab, _SLAB)])


def _stage_out(shared, hbm_slice, stage, slab):
    pltpu.sync_copy(shared.at[pl.ds(slab, _SLAB)], stage)
    pltpu.sync_copy(stage, hbm_slice)


def _sc_degree(dst0, dst1, zeros_np):
    @functools.partial(
        pl.kernel,
        out_type=jax.ShapeDtypeStruct((2, 2, _NP), jnp.float32),
        mesh=_mesh(),
        scratch_types=[
            pltpu.VMEM((_CHE,), jnp.int32),
            pltpu.VMEM((_CHE,), jnp.int32),
            pltpu.VMEM((_CHE,), jnp.float32),
            pltpu.VMEM((_SLAB,), jnp.float32),
            pltpu.SemaphoreType.DMA,
            pltpu.SemaphoreType.DMA,
            pltpu.SemaphoreType.DMA,
            pltpu.SemaphoreType.DMA,
            pltpu.VMEM_SHARED((_NP,), jnp.float32),
            pltpu.VMEM_SHARED((_NP,), jnp.float32),
        ],
    )
    def k(d0_h, d1_h, z_h, out_h, idx0_v, idx1_v, ones_v, stage_v,
          semA, semB, semC, semD, acc0_sh, acc1_sh):
        c = lax.axis_index("c")
        s = lax.axis_index("s")
        slab = s * _SLAB
        pltpu.sync_copy(z_h.at[pl.ds(slab, _SLAB)], stage_v)
        pltpu.sync_copy(stage_v, acc0_sh.at[pl.ds(slab, _SLAB)])
        pltpu.sync_copy(stage_v, acc1_sh.at[pl.ds(slab, _SLAB)])
        for l in range(_CHE // 16):
            ones_v[pl.ds(l * 16, 16)] = jnp.full((16,), 1.0, jnp.float32)
        plsc.subcore_barrier()
        e0 = (s * 2 + c) * _EPT

        def body(kk, carry):
            base = e0 + kk * _CHE
            i0 = pltpu.async_copy(d0_h.at[pl.ds(base, _CHE)], idx0_v, semA)
            i1 = pltpu.async_copy(d1_h.at[pl.ds(base, _CHE)], idx1_v, semB)
            i0.wait()
            s0 = pltpu.async_copy(ones_v, acc0_sh.at[idx0_v], semC, add=True)
            i1.wait()
            s1 = pltpu.async_copy(ones_v, acc1_sh.at[idx1_v], semD, add=True)
            s0.wait()
            s1.wait()
            return carry
        lax.fori_loop(0, _NCH, body, 0)
        plsc.subcore_barrier()
        _stage_out(acc0_sh, out_h.at[0, c, pl.ds(slab, _SLAB)], stage_v, slab)
        _stage_out(acc1_sh, out_h.at[1, c, pl.ds(slab, _SLAB)], stage_v, slab)

    return k(dst0, dst1, zeros_np)


def _sc_gsum(src0, dst0, src1, dst1, y0, y1, zeros_np):
    @functools.partial(
        pl.kernel,
        out_type=jax.ShapeDtypeStruct((2, 2, _NP), jnp.float32),
        mesh=_mesh(),
        scratch_types=[
            pltpu.VMEM((_CHE,), jnp.int32),
            pltpu.VMEM((_CHE,), jnp.int32),
            pltpu.VMEM((_CHE,), jnp.int32),
            pltpu.VMEM((_CHE,), jnp.int32),
            pltpu.VMEM((_CHE,), jnp.float32),
            pltpu.VMEM((_CHE,), jnp.float32),
            pltpu.VMEM((_SLAB,), jnp.float32),
            pltpu.SemaphoreType.DMA,
            pltpu.SemaphoreType.DMA,
            pltpu.SemaphoreType.DMA,
            pltpu.SemaphoreType.DMA,
            pltpu.SemaphoreType.DMA,
            pltpu.SemaphoreType.DMA,
            pltpu.VMEM_SHARED((_NP,), jnp.float32),
            pltpu.VMEM_SHARED((_NP,), jnp.float32),
            pltpu.VMEM_SHARED((_NP,), jnp.float32),
            pltpu.VMEM_SHARED((_NP,), jnp.float32),
        ],
    )
    def k(s0_h, d0_h, s1_h, d1_h, y0_h, y1_h, z_h, out_h,
          idxs0_v, idxd0_v, idxs1_v, idxd1_v, ybuf0_v, ybuf1_v, stage_v,
          semA, semB, semC, semD, semE, semF,
          y0_sh, y1_sh, acc0_sh, acc1_sh):
        c = lax.axis_index("c")
        s = lax.axis_index("s")
        slab = s * _SLAB
        pltpu.sync_copy(z_h.at[pl.ds(slab, _SLAB)], stage_v)
        pltpu.sync_copy(stage_v, acc0_sh.at[pl.ds(slab, _SLAB)])
        pltpu.sync_copy(stage_v, acc1_sh.at[pl.ds(slab, _SLAB)])
        _stage_in(y0_h, y0_sh, stage_v, slab)
        _stage_in(y1_h, y1_sh, stage_v, slab)
        plsc.subcore_barrier()
        e0 = (s * 2 + c) * _EPT

        def body(kk, carry):
            base = e0 + kk * _CHE
            i0a = pltpu.async_copy(s0_h.at[pl.ds(base, _CHE)], idxs0_v, semA)
            i0b = pltpu.async_copy(d0_h.at[pl.ds(base, _CHE)], idxd0_v, semA)
            i1a = pltpu.async_copy(s1_h.at[pl.ds(base, _CHE)], idxs1_v, semB)
            i1b = pltpu.async_copy(d1_h.at[pl.ds(base, _CHE)], idxd1_v, semB)
            i0a.wait()
            i0b.wait()
            g0 = pltpu.async_copy(y0_sh.at[idxs0_v], ybuf0_v, semC)
            i1a.wait()
            i1b.wait()
            g1 = pltpu.async_copy(y1_sh.at[idxs1_v], ybuf1_v, semD)
            g0.wait()
            s0 = pltpu.async_copy(ybuf0_v, acc0_sh.at[idxd0_v], semE, add=True)
            g1.wait()
            s1 = pltpu.async_copy(ybuf1_v, acc1_sh.at[idxd1_v], semF, add=True)
            s0.wait()
            s1.wait()
            return carry
        lax.fori_loop(0, _NCH, body, 0)
        plsc.subcore_barrier()
        _stage_out(acc0_sh, out_h.at[0, c, pl.ds(slab, _SLAB)], stage_v, slab)
        _stage_out(acc1_sh, out_h.at[1, c, pl.ds(slab, _SLAB)], stage_v, slab)

    return k(src0, dst0, src1, dst1, y0, y1, zeros_np)


def _sc_att(src0, dst0, src1, dst1, as0, as1, ad0, ad1, g0, g1, zeros_np):
    @functools.partial(
        pl.kernel,
        out_type=[jax.ShapeDtypeStruct((2, 2, _NP), jnp.float32),
                  jax.ShapeDtypeStruct((2, 2, _NP), jnp.float32)],
        mesh=_mesh(),
        scratch_types=[
            pltpu.VMEM((_CHE,), jnp.int32),
            pltpu.VMEM((_CHE,), jnp.int32),
            pltpu.VMEM((_CHE,), jnp.int32),
            pltpu.VMEM((_CHE,), jnp.int32),
            pltpu.VMEM((_CHE,), jnp.float32),
            pltpu.VMEM((_CHE,), jnp.float32),
            pltpu.VMEM((_CHE,), jnp.float32),
            pltpu.VMEM((_CHE,), jnp.float32),
            pltpu.VMEM((_CHE,), jnp.float32),
            pltpu.VMEM((_CHE,), jnp.float32),
            pltpu.VMEM((_CHE,), jnp.float32),
            pltpu.VMEM((_CHE,), jnp.float32),
            pltpu.VMEM((_CHE,), jnp.float32),
            pltpu.VMEM((_CHE,), jnp.float32),
            pltpu.VMEM((_SLAB,), jnp.float32),
            pltpu.SemaphoreType.DMA,
            pltpu.SemaphoreType.DMA,
            pltpu.SemaphoreType.DMA,
            pltpu.SemaphoreType.DMA,
            pltpu.SemaphoreType.DMA,
            pltpu.SemaphoreType.DMA,
            pltpu.VMEM_SHARED((_NP,), jnp.float32),
            pltpu.VMEM_SHARED((_NP,), jnp.float32),
            pltpu.VMEM_SHARED((_NP,), jnp.float32),
            pltpu.VMEM_SHARED((_NP,), jnp.float32),
            pltpu.VMEM_SHARED((_NP,), jnp.float32),
            pltpu.VMEM_SHARED((_NP,), jnp.float32),
            pltpu.VMEM_SHARED((_NP,), jnp.float32),
            pltpu.VMEM_SHARED((_NP,), jnp.float32),
            pltpu.VMEM_SHARED((_NP,), jnp.float32),
            pltpu.VMEM_SHARED((_NP,), jnp.float32),
        ],
    )
    def k(s0_h, d0_h, s1_h, d1_h, as0_h, as1_h, ad0_h, ad1_h, g0_h, g1_h, z_h,
          zout_h, nout_h,
          idxs0_v, idxd0_v, idxs1_v, idxd1_v,
          asb0_v, adb0_v, gbf0_v, tb0_v, ub0_v,
          asb1_v, adb1_v, gbf1_v, tb1_v, ub1_v, stage_v,
          semA, semB, semC, semD, semE, semF,
          as0_sh, as1_sh, ad0_sh, ad1_sh, g0_sh, g1_sh,
          zac0_sh, zac1_sh, nac0_sh, nac1_sh):
        c = lax.axis_index("c")
        s = lax.axis_index("s")
        slab = s * _SLAB
        pltpu.sync_copy(z_h.at[pl.ds(slab, _SLAB)], stage_v)
        for acc in (zac0_sh, zac1_sh, nac0_sh, nac1_sh):
            pltpu.sync_copy(stage_v, acc.at[pl.ds(slab, _SLAB)])
        for hbm, sh in ((as0_h, as0_sh), (as1_h, as1_sh), (ad0_h, ad0_sh),
                        (ad1_h, ad1_sh), (g0_h, g0_sh), (g1_h, g1_sh)):
            _stage_in(hbm, sh, stage_v, slab)
        plsc.subcore_barrier()
        e0 = (s * 2 + c) * _EPT

        def compute(asb, adb, gbf, tb, ub):
            for l in range(_CHE // 16):
                sl = pl.ds(l * 16, 16)
                e = asb[sl] + adb[sl]
                e = jnp.where(e > 0, e, e * 0.2)
                t = jnp.exp(e)
                tb[sl] = t
                ub[sl] = t * gbf[sl]

        def body(kk, carry):
            base = e0 + kk * _CHE
            i0a = pltpu.async_copy(s0_h.at[pl.ds(base, _CHE)], idxs0_v, semA)
            i0b = pltpu.async_copy(d0_h.at[pl.ds(base, _CHE)], idxd0_v, semA)
            i1a = pltpu.async_copy(s1_h.at[pl.ds(base, _CHE)], idxs1_v, semB)
            i1b = pltpu.async_copy(d1_h.at[pl.ds(base, _CHE)], idxd1_v, semB)
            i0a.wait()
            i0b.wait()
            g0a = pltpu.async_copy(as0_sh.at[idxs0_v], asb0_v, semC)
            g0b = pltpu.async_copy(ad0_sh.at[idxd0_v], adb0_v, semC)
            g0c = pltpu.async_copy(g0_sh.at[idxs0_v], gbf0_v, semC)
            i1a.wait()
            i1b.wait()
            g1a = pltpu.async_copy(as1_sh.at[idxs1_v], asb1_v, semD)
            g1b = pltpu.async_copy(ad1_sh.at[idxd1_v], adb1_v, semD)
            g1c = pltpu.async_copy(g1_sh.at[idxs1_v], gbf1_v, semD)
            g0a.wait()
            g0b.wait()
            g0c.wait()
            compute(asb0_v, adb0_v, gbf0_v, tb0_v, ub0_v)
            s0a = pltpu.async_copy(tb0_v, zac0_sh.at[idxd0_v], semE, add=True)
            s0b = pltpu.async_copy(ub0_v, nac0_sh.at[idxd0_v], semE, add=True)
            g1a.wait()
            g1b.wait()
            g1c.wait()
            compute(asb1_v, adb1_v, gbf1_v, tb1_v, ub1_v)
            s1a = pltpu.async_copy(tb1_v, zac1_sh.at[idxd1_v], semF, add=True)
            s1b = pltpu.async_copy(ub1_v, nac1_sh.at[idxd1_v], semF, add=True)
            s0a.wait()
            s0b.wait()
            s1a.wait()
            s1b.wait()
            return carry
        lax.fori_loop(0, _NCH, body, 0)
        plsc.subcore_barrier()
        _stage_out(zac0_sh, zout_h.at[0, c, pl.ds(slab, _SLAB)], stage_v, slab)
        _stage_out(zac1_sh, zout_h.at[1, c, pl.ds(slab, _SLAB)], stage_v, slab)
        _stage_out(nac0_sh, nout_h.at[0, c, pl.ds(slab, _SLAB)], stage_v, slab)
        _stage_out(nac1_sh, nout_h.at[1, c, pl.ds(slab, _SLAB)], stage_v, slab)

    return k(src0, dst0, src1, dst1, as0, as1, ad0, ad1, g0, g1, zeros_np)


def _tc_prep(x2, degp):
    def body(x_ref, d_ref, y_ref, di_ref):
        d = d_ref[:, 0] + d_ref[:, 1] + 1.0
        di = lax.rsqrt(d)
        di_ref[...] = di
        y_ref[...] = x_ref[...] * di

    return pl.pallas_call(
        body,
        out_shape=[jax.ShapeDtypeStruct((2, _NPR, 128), jnp.float32),
                   jax.ShapeDtypeStruct((2, _NPR, 128), jnp.float32)],
    )(x2, degp)


def _tc_node(gsump, y2, di2, scal):
    def body(gp_ref, y_ref, di_ref, sc_ref, as_ref, ad_ref, g_ref, ts_ref):
        ps, qs, pd, qd = sc_ref[0], sc_ref[1], sc_ref[2], sc_ref[3]
        g = di_ref[...] * (y_ref[...] + gp_ref[:, 0] + gp_ref[:, 1])
        a_s = g * ps + qs
        a_d = g * pd + qd
        e = a_s + a_d
        e = jnp.where(e > 0, e, e * 0.2)
        as_ref[...] = a_s
        ad_ref[...] = a_d
        g_ref[...] = g
        ts_ref[...] = jnp.exp(e)

    return pl.pallas_call(
        body,
        in_specs=[pl.BlockSpec(memory_space=pltpu.MemorySpace.VMEM),
                  pl.BlockSpec(memory_space=pltpu.MemorySpace.VMEM),
                  pl.BlockSpec(memory_space=pltpu.MemorySpace.VMEM),
                  pl.BlockSpec(memory_space=pltpu.MemorySpace.SMEM)],
        out_shape=[jax.ShapeDtypeStruct((2, _NPR, 128), jnp.float32)] * 4,
    )(gsump, y2, di2, scal)


def _tc_final(ts2, g2, z0, z1, n0, n1, ids2, wcols):
    p1, q1, r1, pg, qg, rg, W2, b2, W3, b3, cW1, cb1, cW2, cb2, cW3, cb3 = wcols
    nsteps = 2 * _NBLK

    def body(ts_ref, g_ref, z0_ref, z1_ref, n0_ref, n1_ref, id_ref,
             p1_ref, q1_ref, r1_ref, pg_ref, qg_ref, rg_ref,
             W2_ref, b2_ref, W3_ref, b3_ref,
             cW1_ref, cb1_ref, cW2_ref, cb2_ref, cW3_ref, cb3_ref,
             out_ref, accs, acct):
        i = pl.program_id(0)

        @pl.when(i == 0)
        def _init():
            accs[...] = jnp.zeros((56, 128), jnp.float32)
            acct[...] = jnp.zeros((56, 128), jnp.float32)

        ts = ts_ref[0].reshape(1, 4096)
        g = g_ref[0].reshape(1, 4096)
        z = ts + z0_ref[0].reshape(1, 4096) + z1_ref[0].reshape(1, 4096)
        nm = ts * g + n0_ref[0].reshape(1, 4096) + n1_ref[0].reshape(1, 4096)
        A = nm / (z + 1e-16)
        S = z / (z + 1e-16)
        st = A * p1_ref[...] + S * q1_ref[...] + r1_ref[...]
        st = jnp.where(st > 0, st, st * 0.01)
        gt = A * pg_ref[...] + S * qg_ref[...] + rg_ref[...]
        gt = gt - jnp.max(gt, axis=0, keepdims=True)
        gt = jnp.exp(gt)
        gt = gt / jnp.sum(gt, axis=0, keepdims=True)
        sg = st * gt
        s48 = lax.dot_general(W2_ref[...], sg, (((0,), (0,)), ((), ())),
                              preferred_element_type=jnp.float32)
        s48 = s48 + b2_ref[...]
        s48 = jnp.where(s48 > 0, s48, s48 * 0.01)
        s49 = jnp.concatenate([s48, jnp.ones((1, 4096), jnp.float32)], axis=0)
        ids = id_ref[0].reshape(1, 4096)
        bvec = lax.broadcasted_iota(jnp.int32, (_B, 1), 0)
        oh = (ids == bvec).astype(jnp.float32)
        contrib = lax.dot_general(s49, oh, (((1,), (1,)), ((), ())),
                                  preferred_element_type=jnp.float32)

        @pl.when(i < _NBLK)
        def _adds():
            accs[0:48, :] = accs[0:48, :] + contrib[0:48, :]
            accs[48:56, :] = accs[48:56, :] + contrib[48:49, :]

        @pl.when(i >= _NBLK)
        def _addt():
            acct[0:48, :] = acct[0:48, :] + contrib[0:48, :]
            acct[48:56, :] = acct[48:56, :] + contrib[48:49, :]

        @pl.when(i == nsteps - 1)
        def _cls():
            def emb(acc):
                cnt = acc[48:56, :][0:1, :]
                mean = acc[0:48, :] / jnp.maximum(cnt, 1.0)
                e = lax.dot_general(W3_ref[...], mean, (((0,), (0,)), ((), ())),
                                    preferred_element_type=jnp.float32)
                e = e + b3_ref[...]
                return jnp.where(cnt > 0, e, 0.0)

            cat = jnp.concatenate([emb(accs[...]), emb(acct[...])], axis=0)
            h = lax.dot_general(cW1_ref[...], cat, (((0,), (0,)), ((), ())),
                                preferred_element_type=jnp.float32) + cb1_ref[...]
            h = jnp.maximum(h, 0.0)
            h = lax.dot_general(cW2_ref[...], h, (((0,), (0,)), ((), ())),
                                preferred_element_type=jnp.float32) + cb2_ref[...]
            h = jnp.maximum(h, 0.0)
            o = lax.dot_general(cW3_ref[...], h, (((0,), (0,)), ((), ())),
                                preferred_element_type=jnp.float32) + cb3_ref[...]
            out_ref[...] = o

    node_spec = pl.BlockSpec((1, 32, 128), lambda i: (i, 0, 0))
    full = lambda shape: pl.BlockSpec(shape, lambda i: tuple(0 for _ in shape))
    return pl.pallas_call(
        body,
        grid=(nsteps,),
        in_specs=[node_spec, node_spec, node_spec, node_spec, node_spec,
                  node_spec, node_spec,
                  full((32, 1)), full((32, 1)), full((32, 1)),
                  full((32, 1)), full((32, 1)), full((32, 1)),
                  full((32, 48)), full((48, 1)), full((48, 32)), full((32, 1)),
                  full((64, 32)), full((32, 1)), full((32, 32)), full((32, 1)),
                  full((32, 2)), full((2, 1))],
        out_specs=pl.BlockSpec((2, 128), lambda i: (0, 0)),
        out_shape=jax.ShapeDtypeStruct((2, 128), jnp.float32),
        scratch_shapes=[pltpu.VMEM((56, 128), jnp.float32),
                        pltpu.VMEM((56, 128), jnp.float32)],
    )(ts2, g2, z0, z1, n0, n1, ids2,
      p1, q1, r1, pg, qg, rg, W2, b2, W3, b3, cW1, cb1, cW2, cb2, cW3, cb3)


def _pad_nodes(v):
    return jnp.concatenate([v, jnp.zeros((_NP - _N,), v.dtype)])


def _pad_edges(v):
    return jnp.concatenate(
        [v.astype(jnp.int32),
         jnp.full((_EP - _E,), _SENT, jnp.int32)])


def kernel(x_s, x_t, edge_index_s, edge_index_t, x_s_batch, x_t_batch,
           gcn_W, gcn_b, gat_W, gat_asrc, gat_adst, gat_b,
           agg_W1, agg_b1, agg_Wg, agg_bg, agg_W2, agg_b2, agg_W3, agg_b3,
           cls_W1, cls_b1, cls_W2, cls_b2, cls_W3, cls_b3):
    f32 = jnp.float32
    w = gcn_W[0]
    u = w @ gat_W
    cv = gcn_b @ gat_W
    scal = jnp.stack([u @ gat_asrc, cv @ gat_asrc,
                      u @ gat_adst, cv @ gat_adst]).astype(f32)
    col = lambda v: v.astype(f32)[:, None]
    wcols = (col(u @ agg_W1), col(cv @ agg_W1), col(gat_b @ agg_W1 + agg_b1),
             col(u @ agg_Wg), col(cv @ agg_Wg), col(gat_b @ agg_Wg + agg_bg),
             agg_W2.astype(f32), col(agg_b2), agg_W3.astype(f32), col(agg_b3),
             cls_W1.astype(f32), col(cls_b1), cls_W2.astype(f32), col(cls_b2),
             cls_W3.astype(f32), col(cls_b3))

    x2 = jnp.stack([_pad_nodes(x_s[:, 0]), _pad_nodes(x_t[:, 0])])
    x2 = x2.reshape(2, _NPR, 128)
    src0 = _pad_edges(edge_index_s[0])
    dst0 = _pad_edges(edge_index_s[1])
    src1 = _pad_edges(edge_index_t[0])
    dst1 = _pad_edges(edge_index_t[1])
    pad_ids = jnp.full((_NP - _N,), _B, jnp.int32)
    ids2 = jnp.stack([
        jnp.concatenate([x_s_batch.astype(jnp.int32), pad_ids]),
        jnp.concatenate([x_t_batch.astype(jnp.int32), pad_ids]),
    ]).reshape(2 * _NBLK, 32, 128)
    zeros_np = jnp.zeros((_NP,), f32)

    degp = _sc_degree(dst0, dst1, zeros_np)
    y2, di2 = _tc_prep(x2, degp.reshape(2, 2, _NPR, 128))
    y0 = y2[0].reshape(_NP)
    y1 = y2[1].reshape(_NP)
    gsump = _sc_gsum(src0, dst0, src1, dst1, y0, y1, zeros_np)
    as2, ad2, g2, ts2 = _tc_node(gsump.reshape(2, 2, _NPR, 128), y2, di2, scal)
    zp, nump = _sc_att(src0, dst0, src1, dst1,
                       as2[0].reshape(_NP), as2[1].reshape(_NP),
                       ad2[0].reshape(_NP), ad2[1].reshape(_NP),
                       g2[0].reshape(_NP), g2[1].reshape(_NP), zeros_np)
    blk = lambda a: a.reshape(2 * _NBLK, 32, 128)
    out = _tc_final(blk(ts2), blk(g2),
                    blk(zp[:, 0]), blk(zp[:, 1]),
                    blk(nump[:, 0]), blk(nump[:, 1]), ids2, wcols)
    return out.T
